# baseline (speedup 1.0000x reference)
import functools

import jax
import jax.numpy as jnp
from jax.experimental import pallas as pl
from jax.experimental.pallas import tpu as pltpu

_T = 0.15
_EPS = 1e-8
_BM = 1024
_BN = 4096
_CH = 512
_BP = 1024
_FL = 128


def _prep_kernel(x_ref, y_ref, w_ref, *, nxblk):
    b = pl.program_id(0)
    zb = jnp.where(b < nxblk, x_ref[...], y_ref[...])
    s2 = jnp.sum(zb * zb, axis=1, keepdims=True)
    norm = jnp.maximum(jnp.sqrt(s2), _EPS)
    scale = (1.4426950408889634 / _T) ** 0.5
    w_ref[...] = (zb * (scale / norm)).astype(w_ref.dtype)


def _pair_kernel(wa_ref, wb_ref, s_ref, p_ref, d_ref, ms_ref, o_ref,
                 mrs_ref, l1_ref):
    t = pl.program_id(0)
    sub = _BN // _BM
    nchunk = _BM // _CH

    @pl.when((t == 0) | (t == 8))
    def _():
        s_ref[...] = jnp.zeros_like(s_ref)
        d_ref[...] = jnp.zeros_like(d_ref)

    @pl.when(t == 0)
    def _():
        p_ref[...] = jnp.zeros_like(p_ref)

    b = wb_ref[...]
    rowg = jax.lax.broadcasted_iota(jnp.int32, (_CH, _CH), 0)
    colg = jax.lax.broadcasted_iota(jnp.int32, (_CH, _CH), 1)
    msk = rowg == colg
    ets = []
    for ci, cm in enumerate(range(0, _BM, _CH)):
        a_c = wa_ref[cm:cm + _CH, :]
        ets.append(jnp.exp2(jax.lax.dot_general(
            a_c, b, (((1,), (1,)), ((), ())),
            preferred_element_type=jnp.float32)))

    for ci, cm in enumerate(range(0, _BM, _CH)):
        et = ets[ci]
        s_ref[...] += jnp.sum(et, axis=0, keepdims=True)

        @pl.when((t >= 4) & (t < 8))
        def _():
            part = et[:, 0:_FL]
            for lo in range(_FL, _BN, _FL):
                part = part + et[:, lo:lo + _FL]
            slot = (t - 4) * nchunk + ci
            mrs_ref[pl.ds(slot, 1), :, :] = part.reshape(1, _CH, _FL)

        for r in range(sub):
            lo = r * _BM + cm
            sl = et[:, lo:lo + _CH]

            @pl.when((t == r) | (t == 8 + r))
            def _():
                d_ref[:, lo:lo + _CH] += jnp.sum(
                    jnp.where(msk, sl, 0.0), axis=0, keepdims=True)

            @pl.when(t == 4 + r)
            def _():
                p_ref[:, lo:lo + _CH] += jnp.sum(
                    jnp.where(msk, sl, 0.0), axis=0, keepdims=True)

    @pl.when(t == 7)
    def _():
        allp = mrs_ref[...].reshape(_BN, _FL)
        ones = jnp.ones((1, _FL), jnp.float32)
        ms_ref[...] = jax.lax.dot_general(
            ones, allp, (((1,), (1,)), ((), ())),
            preferred_element_type=jnp.float32)
        p = p_ref[...]
        neg1 = s_ref[...] - d_ref[...] - p
        l1_ref[0, 0] = (jnp.sum(jnp.log(neg1))
                        - 2.0 * jnp.sum(jnp.log(p)))

    @pl.when(t == 11)
    def _():
        neg2 = s_ref[...] + ms_ref[...] - d_ref[...] - p_ref[...]
        o_ref[...] = (l1_ref[0, 0] + jnp.sum(jnp.log(neg2))).reshape(1, 1)


def kernel(x, y):
    n, dm = x.shape
    m = 2 * n
    nxblk = n // _BP
    nsteps = 12

    w = pl.pallas_call(
        functools.partial(_prep_kernel, nxblk=nxblk),
        grid=(m // _BP,),
        in_specs=[
            pl.BlockSpec((_BP, dm), lambda b: (jnp.minimum(b, nxblk - 1), 0)),
            pl.BlockSpec((_BP, dm),
                         lambda b: (jnp.maximum(b - nxblk, 0), 0)),
        ],
        out_specs=pl.BlockSpec((_BP, dm), lambda b: (b, 0)),
        out_shape=jax.ShapeDtypeStruct((m, dm), jnp.float8_e4m3fn),
        compiler_params=pltpu.CompilerParams(
            dimension_semantics=("arbitrary",)),
    )(x, y)

    i_of = lambda t: t // 8
    j_of = lambda t: jnp.where(t < 8, t, t - 4)
    s, p, d, ms, out = pl.pallas_call(
        _pair_kernel,
        grid=(nsteps,),
        in_specs=[
            pl.BlockSpec((_BM, dm), lambda t: (j_of(t), 0)),
            pl.BlockSpec((_BN, dm), lambda t: (i_of(t), 0)),
        ],
        out_specs=[
            pl.BlockSpec((1, _BN), lambda t: (0, i_of(t))),
            pl.BlockSpec((1, n), lambda t: (0, 0)),
            pl.BlockSpec((1, _BN), lambda t: (0, i_of(t))),
            pl.BlockSpec((1, n), lambda t: (0, 0)),
            pl.BlockSpec((1, 1), lambda t: (0, 0)),
        ],
        out_shape=[
            jax.ShapeDtypeStruct((1, m), jnp.float32),
            jax.ShapeDtypeStruct((1, n), jnp.float32),
            jax.ShapeDtypeStruct((1, m), jnp.float32),
            jax.ShapeDtypeStruct((1, n), jnp.float32),
            jax.ShapeDtypeStruct((1, 1), jnp.float32),
        ],
        scratch_shapes=[
            pltpu.VMEM((4 * (_BM // _CH), _CH, _FL), jnp.float32),
            pltpu.SMEM((1, 1), jnp.float32),
        ],
        compiler_params=pltpu.CompilerParams(
            dimension_semantics=("arbitrary",),
            vmem_limit_bytes=62 * 1024 * 1024),
    )(w, w)
    return out[0, 0]
